# Initial kernel scaffold; baseline (speedup 1.0000x reference)
import jax
import jax.numpy as jnp
from jax.experimental import pallas as pl


def kernel(node_features, uniform_noise, sample_weights):
    raise NotImplementedError("write your pallas kernel here")


# speedup vs baseline: 2.0538x; 2.0538x over previous
import functools

import jax
import jax.numpy as jnp
from jax.experimental import pallas as pl
from jax.experimental.pallas import tpu as pltpu

NUM_DIVISION = 10
NUM_NODES = 100000
D_FEAT = 128
TAU = 1.0
CHUNK = NUM_NODES // NUM_DIVISION

BLOCK_ROWS = 2000
NUM_BLOCKS = NUM_NODES // BLOCK_ROWS
BLOCKS_PER_CHUNK = CHUNK // BLOCK_ROWS


def _copy_scale_kernel(idx_ref, scale_ref, x_ref, out_ref, outidx_ref):
    i = pl.program_id(0)

    @pl.when(i == 0)
    def _():
        base = idx_ref[0] * CHUNK
        outidx_ref[...] = base + jax.lax.broadcasted_iota(
            jnp.int32, (1, CHUNK), 1
        )

    in_slab = (i // BLOCKS_PER_CHUNK) == idx_ref[0]
    w = jnp.where(in_slab, scale_ref[0], jnp.float32(1.0))
    out_ref[...] = x_ref[...] * w


@functools.partial(jax.jit, static_argnames=("interpret",))
def kernel(node_features, uniform_noise, sample_weights, interpret=False):
    g = -jnp.log(-jnp.log(uniform_noise))
    y_soft = jax.nn.softmax((sample_weights + g) / TAU, axis=-1)
    idx = jnp.argmax(y_soft, axis=-1).astype(jnp.int32)
    y = (1.0 + y_soft[idx]) - y_soft[idx]
    idx_arr = idx[None]
    scale_arr = y[None].astype(jnp.float32)

    updated, outidx = pl.pallas_call(
        _copy_scale_kernel,
        grid=(NUM_BLOCKS,),
        in_specs=[
            pl.BlockSpec(memory_space=pltpu.SMEM),
            pl.BlockSpec(memory_space=pltpu.SMEM),
            pl.BlockSpec((BLOCK_ROWS, D_FEAT), lambda i: (i, 0)),
        ],
        out_specs=[
            pl.BlockSpec((BLOCK_ROWS, D_FEAT), lambda i: (i, 0)),
            pl.BlockSpec((1, CHUNK), lambda i: (0, 0)),
        ],
        out_shape=[
            jax.ShapeDtypeStruct((NUM_NODES, D_FEAT), jnp.float32),
            jax.ShapeDtypeStruct((1, CHUNK), jnp.int32),
        ],
        compiler_params=pltpu.CompilerParams(
            dimension_semantics=("arbitrary",),
        ),
        interpret=interpret,
    )(idx_arr, scale_arr, node_features)

    return updated, outidx.reshape(CHUNK)


# speedup vs baseline: 2.7447x; 1.3364x over previous
import functools

import jax
import jax.numpy as jnp
from jax.experimental import pallas as pl
from jax.experimental.pallas import tpu as pltpu

NUM_DIVISION = 10
NUM_NODES = 100000
D_FEAT = 128
TAU = 1.0
CHUNK = NUM_NODES // NUM_DIVISION

BLOCK_ROWS = 5000
NUM_BLOCKS = NUM_NODES // BLOCK_ROWS
BLOCKS_PER_CHUNK = CHUNK // BLOCK_ROWS


def _copy_scale_kernel(idx_ref, scale_ref, x_ref, out_ref, outidx_ref):
    i = pl.program_id(0)

    @pl.when(i == 0)
    def _():
        base = idx_ref[0] * CHUNK
        outidx_ref[...] = base + jax.lax.broadcasted_iota(
            jnp.int32, (1, CHUNK), 1
        )

    in_slab = (i // BLOCKS_PER_CHUNK) == idx_ref[0]
    w = jnp.where(in_slab, scale_ref[0], jnp.float32(1.0))
    out_ref[...] = x_ref[...] * w


@functools.partial(jax.jit, static_argnames=("interpret",))
def kernel(node_features, uniform_noise, sample_weights, interpret=False):
    g = -jnp.log(-jnp.log(uniform_noise))
    y_soft = jax.nn.softmax((sample_weights + g) / TAU, axis=-1)
    idx = jnp.argmax(y_soft, axis=-1).astype(jnp.int32)
    y = (1.0 + y_soft[idx]) - y_soft[idx]
    idx_arr = idx[None]
    scale_arr = y[None].astype(jnp.float32)

    updated, outidx = pl.pallas_call(
        _copy_scale_kernel,
        grid=(NUM_BLOCKS,),
        in_specs=[
            pl.BlockSpec(memory_space=pltpu.SMEM),
            pl.BlockSpec(memory_space=pltpu.SMEM),
            pl.BlockSpec((BLOCK_ROWS, D_FEAT), lambda i: (i, 0)),
        ],
        out_specs=[
            pl.BlockSpec((BLOCK_ROWS, D_FEAT), lambda i: (i, 0)),
            pl.BlockSpec((1, CHUNK), lambda i: (0, 0)),
        ],
        out_shape=[
            jax.ShapeDtypeStruct((NUM_NODES, D_FEAT), jnp.float32),
            jax.ShapeDtypeStruct((1, CHUNK), jnp.int32),
        ],
        compiler_params=pltpu.CompilerParams(
            dimension_semantics=("arbitrary",),
        ),
        interpret=interpret,
    )(idx_arr, scale_arr, node_features)

    return updated, outidx.reshape(CHUNK)


# speedup vs baseline: 2.8826x; 1.0503x over previous
import functools

import jax
import jax.numpy as jnp
from jax.experimental import pallas as pl
from jax.experimental.pallas import tpu as pltpu

NUM_DIVISION = 10
NUM_NODES = 100000
D_FEAT = 128
TAU = 1.0
CHUNK = NUM_NODES // NUM_DIVISION

BLOCK_ROWS = 10000
NUM_BLOCKS = NUM_NODES // BLOCK_ROWS
BLOCKS_PER_CHUNK = CHUNK // BLOCK_ROWS


def _copy_scale_kernel(idx_ref, scale_ref, x_ref, out_ref, outidx_ref):
    i = pl.program_id(0)

    @pl.when(i == 0)
    def _():
        base = idx_ref[0] * CHUNK
        outidx_ref[...] = base + jax.lax.broadcasted_iota(
            jnp.int32, (1, CHUNK), 1
        )

    in_slab = (i // BLOCKS_PER_CHUNK) == idx_ref[0]
    w = jnp.where(in_slab, scale_ref[0], jnp.float32(1.0))
    out_ref[...] = x_ref[...] * w


@functools.partial(jax.jit, static_argnames=("interpret",))
def kernel(node_features, uniform_noise, sample_weights, interpret=False):
    g = -jnp.log(-jnp.log(uniform_noise))
    y_soft = jax.nn.softmax((sample_weights + g) / TAU, axis=-1)
    idx = jnp.argmax(y_soft, axis=-1).astype(jnp.int32)
    y = (1.0 + y_soft[idx]) - y_soft[idx]
    idx_arr = idx[None]
    scale_arr = y[None].astype(jnp.float32)

    updated, outidx = pl.pallas_call(
        _copy_scale_kernel,
        grid=(NUM_BLOCKS,),
        in_specs=[
            pl.BlockSpec(memory_space=pltpu.SMEM),
            pl.BlockSpec(memory_space=pltpu.SMEM),
            pl.BlockSpec((BLOCK_ROWS, D_FEAT), lambda i: (i, 0)),
        ],
        out_specs=[
            pl.BlockSpec((BLOCK_ROWS, D_FEAT), lambda i: (i, 0)),
            pl.BlockSpec((1, CHUNK), lambda i: (0, 0)),
        ],
        out_shape=[
            jax.ShapeDtypeStruct((NUM_NODES, D_FEAT), jnp.float32),
            jax.ShapeDtypeStruct((1, CHUNK), jnp.int32),
        ],
        compiler_params=pltpu.CompilerParams(
            dimension_semantics=("arbitrary",),
        ),
        interpret=interpret,
    )(idx_arr, scale_arr, node_features)

    return updated, outidx.reshape(CHUNK)
